# Initial kernel scaffold; baseline (speedup 1.0000x reference)
import functools

import jax
import jax.numpy as jnp
from jax import lax
from jax.experimental import pallas as pl
from jax.experimental.pallas import tpu as pltpu
from jax.experimental.pallas import tpu_sc as plsc

NUM_USERS = 50000
NUM_ITEMS = 50000
NN = NUM_USERS + NUM_ITEMS
D = 32
NE = 1600000
TEMP = 0.2
B = 4096

NC = 2
NS = 16
HALF = NN // NC
ACC_ROWS = 50048
ZPT = ACC_ROWS // NS
WPT = HALF // NS
DUMP = ACC_ROWS - 1
EC = 1024
EPT = NE // NS
SUB = 128

mesh = plsc.VectorSubcoreMesh(core_axis_name="c", subcore_axis_name="s")


def _zero_rows(rows):
    z = jnp.zeros((16,), jnp.float32)

    def zb(i, _):
        rows[i, pl.ds(0, 16)] = z
        rows[i, pl.ds(16, 16)] = z
        return 0

    lax.fori_loop(0, EC, zb, 0)


def _layer_body(src_h, dst_h, w_h, emb_h, out_h,
                acc, rows, srcb, wb, locb, idx2, srcv, dstv, wv,
                semg, sems, seme):
    c = lax.axis_index("c")
    s = lax.axis_index("s")
    sc_base = c * HALF

    _zero_rows(rows)
    zb = s * ZPT
    pltpu.sync_copy(rows.at[pl.ds(0, 1024)], acc.at[pl.ds(zb, 1024)])
    pltpu.sync_copy(rows.at[pl.ds(0, 1024)], acc.at[pl.ds(zb + 1024, 1024)])
    pltpu.sync_copy(rows.at[pl.ds(0, 1024)], acc.at[pl.ds(zb + 2048, 1024)])
    pltpu.sync_copy(rows.at[pl.ds(0, 56)], acc.at[pl.ds(zb + 3072, 56)])
    plsc.subcore_barrier()

    def process_chunk(base, n):
        pltpu.async_copy(src_h.at[pl.ds(base, n)], srcv.at[pl.ds(0, n)], seme)
        pltpu.async_copy(dst_h.at[pl.ds(base, n)], dstv.at[pl.ds(0, n)], seme)
        pltpu.async_copy(w_h.at[pl.ds(base, n)], wv.at[pl.ds(0, n)], seme)
        pltpu.make_async_copy(src_h.at[pl.ds(base, n)], srcv.at[pl.ds(0, n)], seme).wait()
        pltpu.make_async_copy(dst_h.at[pl.ds(base, n)], dstv.at[pl.ds(0, n)], seme).wait()
        pltpu.make_async_copy(w_h.at[pl.ds(base, n)], wv.at[pl.ds(0, n)], seme).wait()

        zsrc = jnp.zeros((16,), jnp.int32)
        zw = jnp.zeros((16,), jnp.float32)
        zloc = jnp.full((16,), DUMP, jnp.int32)

        def pf(i, _):
            srcb[pl.ds(i * 16, 16)] = zsrc
            wb[pl.ds(i * 16, 16)] = zw
            locb[pl.ds(i * 16, 16)] = zloc
            return 0

        lax.fori_loop(0, 65, pf, 0)

        def cp(v, off):
            dd = dstv[pl.ds(v * 16, 16)]
            loc = dd - sc_base
            ok = (loc >= 0) & (loc < HALF)
            sv = srcv[pl.ds(v * 16, 16)]
            wvv = wv[pl.ds(v * 16, 16)]
            plsc.store_compressed(srcb.at[pl.ds(off, 16)], sv, ok)
            plsc.store_compressed(wb.at[pl.ds(off, 16)], wvv, ok)
            plsc.store_compressed(locb.at[pl.ds(off, 16)], loc, ok)
            pc = plsc.all_reduce_population_count(ok)
            return off + jnp.max(pc)

        m = lax.fori_loop(0, n // 16, cp, jnp.int32(0))
        nb = (m + (SUB - 1)) // SUB

        for v in range(64):
            idx2[v // 8, pl.ds((v % 8) * 16, 16)] = locb[pl.ds(v * 16, 16)]

        def gf(j, _):
            pltpu.async_copy(emb_h.at[srcb.at[pl.ds(j * SUB, SUB)]],
                             rows.at[pl.ds(j * SUB, SUB)], semg)
            return 0

        lax.fori_loop(0, nb, gf, 0)

        def gw(j, _):
            pltpu.make_async_copy(emb_h.at[srcb.at[pl.ds(j * SUB, SUB)]],
                                  rows.at[pl.ds(j * SUB, SUB)], semg).wait()
            return 0

        lax.fori_loop(0, nb, gw, 0)

        def sc4(e4, _):
            for u in range(4):
                e = e4 * 4 + u
                wsp = plsc.load_gather(wb, [jnp.full((16,), 0, jnp.int32) + e])
                rows[e, pl.ds(0, 16)] = rows[e, pl.ds(0, 16)] * wsp
                rows[e, pl.ds(16, 16)] = rows[e, pl.ds(16, 16)] * wsp
            return 0

        lax.fori_loop(0, (nb * SUB) // 4, sc4, 0)

        def sf(j, _):
            pltpu.async_copy(rows.at[pl.ds(j * SUB, SUB)],
                             acc.at[idx2.at[j]], sems, add=True)
            return 0

        lax.fori_loop(0, nb, sf, 0)

        def sw(j, _):
            pltpu.make_async_copy(rows.at[pl.ds(j * SUB, SUB)],
                                  acc.at[idx2.at[j]], sems).wait()
            return 0

        lax.fori_loop(0, nb, sw, 0)

    ebase = s * EPT
    nfull = EPT // EC

    def ch(k, _):
        process_chunk(ebase + k * EC, EC)
        return 0

    lax.fori_loop(0, nfull, ch, 0)
    process_chunk(ebase + nfull * EC, EPT - nfull * EC)

    plsc.subcore_barrier()

    wbase = s * WPT
    pltpu.sync_copy(acc.at[pl.ds(wbase, WPT)],
                    out_h.at[pl.ds(sc_base + wbase, WPT)])


_layer_call = functools.partial(
    pl.kernel,
    out_type=jax.ShapeDtypeStruct((NN, D), jnp.float32),
    mesh=mesh,
    scratch_types=[
        pltpu.VMEM_SHARED((ACC_ROWS, D), jnp.float32),
        pltpu.VMEM((EC, D), jnp.float32),
        pltpu.VMEM((1040,), jnp.int32),
        pltpu.VMEM((1040,), jnp.float32),
        pltpu.VMEM((1040,), jnp.int32),
        pltpu.VMEM((8, SUB), jnp.int32),
        pltpu.VMEM((EC,), jnp.int32),
        pltpu.VMEM((EC,), jnp.int32),
        pltpu.VMEM((EC,), jnp.float32),
        pltpu.SemaphoreType.DMA,
        pltpu.SemaphoreType.DMA,
        pltpu.SemaphoreType.DMA,
    ],
)(_layer_body)


BPW = B // (NC * NS)


def _gather3_body(u_h, p_h, e0_h, e1_h, e2_h, ue_h, pe_h,
                  idxb, g0, g1, g2, ob, sem):
    c = lax.axis_index("c")
    s = lax.axis_index("s")
    wid = s * NC + c
    base = wid * BPW
    third = jnp.full((16,), 1.0 / 3.0, jnp.float32)

    for which in range(2):
        ih = u_h if which == 0 else p_h
        oh = ue_h if which == 0 else pe_h
        pltpu.sync_copy(ih.at[pl.ds(base, BPW)], idxb)
        if which == 1:
            off = jnp.full((16,), NUM_USERS, jnp.int32)

            def adj(i, _):
                idxb[pl.ds(i * 16, 16)] = idxb[pl.ds(i * 16, 16)] + off
                return 0

            lax.fori_loop(0, BPW // 16, adj, 0)
        pltpu.async_copy(e0_h.at[idxb], g0, sem)
        pltpu.async_copy(e1_h.at[idxb], g1, sem)
        pltpu.async_copy(e2_h.at[idxb], g2, sem)
        pltpu.make_async_copy(e0_h.at[idxb], g0, sem).wait()
        pltpu.make_async_copy(e1_h.at[idxb], g1, sem).wait()
        pltpu.make_async_copy(e2_h.at[idxb], g2, sem).wait()

        def mix(i, _):
            for h in range(2):
                sl = pl.ds(h * 16, 16)
                ob[i, sl] = (g0[i, sl] + g1[i, sl] + g2[i, sl]) * third
            return 0

        lax.fori_loop(0, BPW, mix, 0)
        pltpu.sync_copy(ob, oh.at[pl.ds(base, BPW)])


_gather3_call = functools.partial(
    pl.kernel,
    out_type=(jax.ShapeDtypeStruct((B, D), jnp.float32),
              jax.ShapeDtypeStruct((B, D), jnp.float32)),
    mesh=mesh,
    scratch_types=[
        pltpu.VMEM((BPW,), jnp.int32),
        pltpu.VMEM((BPW, D), jnp.float32),
        pltpu.VMEM((BPW, D), jnp.float32),
        pltpu.VMEM((BPW, D), jnp.float32),
        pltpu.VMEM((BPW, D), jnp.float32),
        pltpu.SemaphoreType.DMA,
    ],
)(_gather3_body)


def _loss_body(ue_ref, pe_ref, uc_ref, pc_ref, out_ref):
    ue = ue_ref[...]
    pe = pe_ref[...]
    eps = jnp.float32(1e-12)
    un = ue / jnp.maximum(jnp.sqrt(jnp.sum(ue * ue, axis=1, keepdims=True)), eps)
    pn = pe / jnp.maximum(jnp.sqrt(jnp.sum(pe * pe, axis=1, keepdims=True)), eps)

    ip = jnp.sum(un * pn, axis=1, keepdims=True)
    up_score = jnp.exp(ip / TEMP) + jnp.exp(ip * ip / TEMP)
    up = jnp.sum(jnp.log(up_score)) / B

    total = jnp.float32(0.0)
    BL = 512
    for j in range(B // BL):
        pj = lax.slice(pn, (j * BL, 0), ((j + 1) * BL, D))
        sim = lax.dot_general(un, pj, (((1,), (1,)), ((), ())),
                              preferred_element_type=jnp.float32)
        total = total + jnp.sum(jnp.exp(sim / TEMP) + jnp.exp(sim * sim / TEMP))

    def distinct(col):
        cnt = jnp.float32(0.0)
        rowfull = col.reshape(1, B)
        CB = 256
        for bi in range(B // CB):
            blk = lax.slice(col, (bi * CB, 0), ((bi + 1) * CB, 1))
            eq = (blk == rowfull)
            jlt = (lax.broadcasted_iota(jnp.int32, (CB, B), 1) <
                   (lax.broadcasted_iota(jnp.int32, (CB, B), 0) + bi * CB))
            dup = jnp.sum(jnp.where(eq & jlt, 1.0, 0.0), axis=1, keepdims=True) > 0
            cnt = cnt + (CB - jnp.sum(jnp.where(dup, 1.0, 0.0)))
        return cnt

    n_u = distinct(uc_ref[...])
    n_i = distinct(pc_ref[...])

    down = jnp.log(total / (n_u * n_i))
    ii = lax.broadcasted_iota(jnp.int32, (8, 128), 0)
    jj = lax.broadcasted_iota(jnp.int32, (8, 128), 1)
    out_ref[...] = (jnp.where((ii == 0) & (jj == 0), -up, 0.0)
                    + jnp.where((ii == 0) & (jj == 1), down, 0.0))


def _loss_call(ue, pe, ucol, pcol):
    return pl.pallas_call(
        _loss_body,
        out_shape=jax.ShapeDtypeStruct((8, 128), jnp.float32),
    )(ue, pe, ucol, pcol)


def kernel(user, positive, negative, user_table, item_table, edge_index, edge_weight):
    emb0 = jnp.concatenate([user_table, item_table], axis=0)
    src = edge_index[0]
    dst = edge_index[1]
    emb1 = _layer_call(src, dst, edge_weight, emb0)
    emb2 = _layer_call(src, dst, edge_weight, emb1)
    ue, pe = _gather3_call(user, positive, emb0, emb1, emb2)
    blk = _loss_call(ue, pe, user.reshape(B, 1), positive.reshape(B, 1))
    return blk[0, :2]


# speedup vs baseline: 2.0263x; 2.0263x over previous
import functools

import jax
import jax.numpy as jnp
from jax import lax
from jax.experimental import pallas as pl
from jax.experimental.pallas import tpu as pltpu
from jax.experimental.pallas import tpu_sc as plsc

NUM_USERS = 50000
NUM_ITEMS = 50000
NN = NUM_USERS + NUM_ITEMS
D = 32
NE = 1600000
TEMP = 0.2
B = 4096

NC = 2
NS = 16
HALF = NN // NC
ACC_ROWS = 50048
ZPT = ACC_ROWS // NS
WPT = HALF // NS
DUMP = ACC_ROWS - 1
EC = 512
EPT = NE // NS
SUB = 128

_MESH_CACHE = []


def _mesh():
    if not _MESH_CACHE:
        _MESH_CACHE.append(plsc.VectorSubcoreMesh(
            core_axis_name="c", subcore_axis_name="s",
            num_cores=NC, num_subcores=NS))
    return _MESH_CACHE[0]


def _zero_rows(rows):
    z = jnp.zeros((16,), jnp.float32)

    def zb(i, _):
        rows[i, pl.ds(0, 16)] = z
        rows[i, pl.ds(16, 16)] = z
        return 0

    lax.fori_loop(0, EC, zb, 0)


def _layer_body(src_h, dst_h, w_h, emb_h, out_h,
                acc, rows, srcb, wb, locb, idx2, srcv, dstv, wv,
                semg, sems, seme):
    c = lax.axis_index("c")
    s = lax.axis_index("s")
    sc_base = c * HALF

    _zero_rows(rows)
    zb = s * ZPT
    for i in range(6):
        pltpu.sync_copy(rows.at[pl.ds(0, 512)], acc.at[pl.ds(zb + i * 512, 512)])
    pltpu.sync_copy(rows.at[pl.ds(0, 56)], acc.at[pl.ds(zb + 3072, 56)])
    plsc.subcore_barrier()

    def process_chunk(base, n):
        pltpu.async_copy(src_h.at[pl.ds(base, n)], srcv.at[pl.ds(0, n)], seme)
        pltpu.async_copy(dst_h.at[pl.ds(base, n)], dstv.at[pl.ds(0, n)], seme)
        pltpu.async_copy(w_h.at[pl.ds(base, n)], wv.at[pl.ds(0, n)], seme)
        pltpu.make_async_copy(src_h.at[pl.ds(base, n)], srcv.at[pl.ds(0, n)], seme).wait()
        pltpu.make_async_copy(dst_h.at[pl.ds(base, n)], dstv.at[pl.ds(0, n)], seme).wait()
        pltpu.make_async_copy(w_h.at[pl.ds(base, n)], wv.at[pl.ds(0, n)], seme).wait()

        zsrc = jnp.zeros((16,), jnp.int32)
        zw = jnp.zeros((16,), jnp.float32)
        zloc = jnp.full((16,), DUMP, jnp.int32)

        def pf(i, _):
            srcb[pl.ds(i * 16, 16)] = zsrc
            wb[pl.ds(i * 16, 16)] = zw
            locb[pl.ds(i * 16, 16)] = zloc
            return 0

        lax.fori_loop(0, (EC + 16) // 16, pf, 0)

        def cp(v, off):
            dd = dstv[pl.ds(v * 16, 16)]
            loc = dd - sc_base
            ok = (loc >= 0) & (loc < HALF)
            sv = srcv[pl.ds(v * 16, 16)]
            wvv = wv[pl.ds(v * 16, 16)]
            inc = plsc.cumsum(jnp.where(ok, 1, 0).astype(jnp.int32))
            pos = jnp.where(ok, off + inc - 1, EC)
            plsc.store_scatter(srcb, [pos], sv)
            plsc.store_scatter(wb, [pos], wvv)
            plsc.store_scatter(locb, [pos], loc)
            return off + jnp.max(inc)

        m = lax.fori_loop(0, n // 16, cp, jnp.int32(0))
        nb = (m + (SUB - 1)) // SUB

        for v in range(EC // 16):
            idx2[v // 8, pl.ds((v % 8) * 16, 16)] = locb[pl.ds(v * 16, 16)]

        def gf(j, _):
            pltpu.async_copy(emb_h.at[srcb.at[pl.ds(j * SUB, SUB)]],
                             rows.at[pl.ds(j * SUB, SUB)], semg)
            return 0

        lax.fori_loop(0, nb, gf, 0)

        def gw(j, _):
            pltpu.make_async_copy(emb_h.at[srcb.at[pl.ds(j * SUB, SUB)]],
                                  rows.at[pl.ds(j * SUB, SUB)], semg).wait()
            return 0

        lax.fori_loop(0, nb, gw, 0)

        def sc4(e4, _):
            for u in range(4):
                e = e4 * 4 + u
                wsp = plsc.load_gather(wb, [jnp.full((16,), 0, jnp.int32) + e])
                rows[e, pl.ds(0, 16)] = rows[e, pl.ds(0, 16)] * wsp
                rows[e, pl.ds(16, 16)] = rows[e, pl.ds(16, 16)] * wsp
            return 0

        lax.fori_loop(0, (nb * SUB) // 4, sc4, 0)

        def sf(j, _):
            pltpu.async_copy(rows.at[pl.ds(j * SUB, SUB)],
                             acc.at[idx2.at[j]], sems, add=True)
            return 0

        lax.fori_loop(0, nb, sf, 0)

        def sw(j, _):
            pltpu.make_async_copy(rows.at[pl.ds(j * SUB, SUB)],
                                  acc.at[idx2.at[j]], sems).wait()
            return 0

        lax.fori_loop(0, nb, sw, 0)

    ebase = s * EPT
    nfull = EPT // EC

    def ch(k, _):
        process_chunk(ebase + k * EC, EC)
        return 0

    lax.fori_loop(0, nfull, ch, 0)
    process_chunk(ebase + nfull * EC, EPT - nfull * EC)

    plsc.subcore_barrier()

    wbase = s * 3128

    @pl.when(s < NS - 1)
    def _():
        pltpu.sync_copy(acc.at[pl.ds(wbase, 3128)],
                        out_h.at[pl.ds(sc_base + wbase, 3128)])

    @pl.when(s == NS - 1)
    def _():
        pltpu.sync_copy(acc.at[pl.ds((NS - 1) * 3128, 3080)],
                        out_h.at[pl.ds(sc_base + (NS - 1) * 3128, 3080)])


def _layer_call(src, dst, w, emb):
    return pl.kernel(
        _layer_body,
        out_type=jax.ShapeDtypeStruct((NN, D), jnp.float32),
        mesh=_mesh(),
        compiler_params=pltpu.CompilerParams(needs_layout_passes=False, use_tc_tiling_on_sc=False),
        scratch_types=[
        pltpu.VMEM_SHARED((ACC_ROWS, D), jnp.float32),
        pltpu.VMEM((EC, D), jnp.float32),
        pltpu.VMEM((EC + 16,), jnp.int32),
        pltpu.VMEM((EC + 16,), jnp.float32),
        pltpu.VMEM((EC + 16,), jnp.int32),
        pltpu.VMEM((EC // SUB, SUB), jnp.int32),
        pltpu.VMEM((EC,), jnp.int32),
        pltpu.VMEM((EC,), jnp.int32),
        pltpu.VMEM((EC,), jnp.float32),
            pltpu.SemaphoreType.DMA,
            pltpu.SemaphoreType.DMA,
            pltpu.SemaphoreType.DMA,
        ],
    )(src, dst, w, emb)


BPW = B // (NC * NS)


def _gather3_body(u_h, p_h, e0_h, e1_h, e2_h, ue_h, pe_h,
                  idxb, g0, g1, g2, ob, sem):
    c = lax.axis_index("c")
    s = lax.axis_index("s")
    wid = s * NC + c
    base = wid * BPW
    third = jnp.full((16,), 1.0 / 3.0, jnp.float32)

    for which in range(2):
        ih = u_h if which == 0 else p_h
        oh = ue_h if which == 0 else pe_h
        pltpu.sync_copy(ih.at[pl.ds(base, BPW)], idxb)
        if which == 1:
            off = jnp.full((16,), NUM_USERS, jnp.int32)

            def adj(i, _):
                idxb[pl.ds(i * 16, 16)] = idxb[pl.ds(i * 16, 16)] + off
                return 0

            lax.fori_loop(0, BPW // 16, adj, 0)
        pltpu.async_copy(e0_h.at[idxb], g0, sem)
        pltpu.async_copy(e1_h.at[idxb], g1, sem)
        pltpu.async_copy(e2_h.at[idxb], g2, sem)
        pltpu.make_async_copy(e0_h.at[idxb], g0, sem).wait()
        pltpu.make_async_copy(e1_h.at[idxb], g1, sem).wait()
        pltpu.make_async_copy(e2_h.at[idxb], g2, sem).wait()

        def mix(i, _):
            for h in range(2):
                sl = pl.ds(h * 16, 16)
                ob[i, sl] = (g0[i, sl] + g1[i, sl] + g2[i, sl]) * third
            return 0

        lax.fori_loop(0, BPW, mix, 0)
        pltpu.sync_copy(ob, oh.at[pl.ds(base, BPW)])


def _gather3_call(user, positive, e0, e1, e2):
    return pl.kernel(
        _gather3_body,
        out_type=(jax.ShapeDtypeStruct((B, D), jnp.float32),
                  jax.ShapeDtypeStruct((B, D), jnp.float32)),
        mesh=_mesh(),
        compiler_params=pltpu.CompilerParams(needs_layout_passes=False, use_tc_tiling_on_sc=False),
        scratch_types=[
            pltpu.VMEM((BPW,), jnp.int32),
            pltpu.VMEM((BPW, D), jnp.float32),
            pltpu.VMEM((BPW, D), jnp.float32),
            pltpu.VMEM((BPW, D), jnp.float32),
            pltpu.VMEM((BPW, D), jnp.float32),
            pltpu.SemaphoreType.DMA,
        ],
    )(user, positive, e0, e1, e2)


def _loss_body(ue_ref, pe_ref, uc_ref, pc_ref, out_ref):
    ue = ue_ref[...]
    pe = pe_ref[...]
    eps = jnp.float32(1e-12)
    un = ue / jnp.maximum(jnp.sqrt(jnp.sum(ue * ue, axis=1, keepdims=True)), eps)
    pn = pe / jnp.maximum(jnp.sqrt(jnp.sum(pe * pe, axis=1, keepdims=True)), eps)

    ip = jnp.sum(un * pn, axis=1, keepdims=True)
    up_score = jnp.exp(ip / TEMP) + jnp.exp(ip * ip / TEMP)
    up = jnp.sum(jnp.log(up_score)) / B

    total = jnp.float32(0.0)
    BL = 512
    for j in range(B // BL):
        pj = lax.slice(pn, (j * BL, 0), ((j + 1) * BL, D))
        sim = lax.dot_general(un, pj, (((1,), (1,)), ((), ())),
                              preferred_element_type=jnp.float32)
        total = total + jnp.sum(jnp.exp(sim / TEMP) + jnp.exp(sim * sim / TEMP))

    def distinct(col):
        cnt = jnp.float32(0.0)
        rowfull = col.reshape(1, B)
        CB = 256
        for bi in range(B // CB):
            blk = lax.slice(col, (bi * CB, 0), ((bi + 1) * CB, 1))
            eq = (blk == rowfull)
            jlt = (lax.broadcasted_iota(jnp.int32, (CB, B), 1) <
                   (lax.broadcasted_iota(jnp.int32, (CB, B), 0) + bi * CB))
            dup = jnp.sum(jnp.where(eq & jlt, 1.0, 0.0), axis=1, keepdims=True) > 0
            cnt = cnt + (CB - jnp.sum(jnp.where(dup, 1.0, 0.0)))
        return cnt

    n_u = distinct(uc_ref[...])
    n_i = distinct(pc_ref[...])

    down = jnp.log(total / (n_u * n_i))
    ii = lax.broadcasted_iota(jnp.int32, (8, 128), 0)
    jj = lax.broadcasted_iota(jnp.int32, (8, 128), 1)
    out_ref[...] = (jnp.where((ii == 0) & (jj == 0), -up, 0.0)
                    + jnp.where((ii == 0) & (jj == 1), down, 0.0))


def _loss_call(ue, pe, ucol, pcol):
    return pl.pallas_call(
        _loss_body,
        out_shape=jax.ShapeDtypeStruct((8, 128), jnp.float32),
    )(ue, pe, ucol, pcol)


def kernel(user, positive, negative, user_table, item_table, edge_index, edge_weight):
    emb0 = jnp.concatenate([user_table, item_table], axis=0)
    src = edge_index[0]
    dst = edge_index[1]
    emb1 = _layer_call(src, dst, edge_weight, emb0)
    emb2 = _layer_call(src, dst, edge_weight, emb1)
    ue, pe = _gather3_call(user, positive, emb0, emb1, emb2)
    blk = _loss_call(ue, pe, user.reshape(B, 1), positive.reshape(B, 1))
    return blk[0, :2]
